# baseline (speedup 1.0000x reference)
import functools

import jax
import jax.numpy as jnp
from jax import lax
from jax.experimental import pallas as pl
from jax.experimental.pallas import tpu as pltpu
from jax.experimental.pallas import tpu_sc as plsc

N = 10000
E = 320000
H = 128

NP = 10240
NC = 1
NS = 16
NW = NC * NS
SUB = 64
KI = 40
NBLK = 8
NSUB = KI * NBLK
EP = NW * NSUB * SUB
ROWS_PER_TILE = NP // NS


def _pre_body(h_ref, wa_ref, wb_ref, be_ref, a_ref, b_ref):
    h = h_ref[...]
    a_ref[...] = jnp.dot(h, wa_ref[...], preferred_element_type=jnp.float32)
    b_ref[...] = (jnp.dot(h, wb_ref[...], preferred_element_type=jnp.float32)
                  + be_ref[...])


def _pre_ab(hidden_pad, wa, wb, be2):
    blk = NP // 5
    return pl.pallas_call(
        _pre_body,
        grid=(5,),
        in_specs=[
            pl.BlockSpec((blk, H), lambda i: (i, 0)),
            pl.BlockSpec((H, H), lambda i: (0, 0)),
            pl.BlockSpec((H, H), lambda i: (0, 0)),
            pl.BlockSpec((1, H), lambda i: (0, 0)),
        ],
        out_specs=[
            pl.BlockSpec((blk, H), lambda i: (i, 0)),
            pl.BlockSpec((blk, H), lambda i: (i, 0)),
        ],
        out_shape=[
            jax.ShapeDtypeStruct((NP, H), jnp.float32),
            jax.ShapeDtypeStruct((NP, H), jnp.float32),
        ],
    )(hidden_pad, wa, wb, be2)


def _zero_buf(buf):
    zeros = jnp.zeros((16,), jnp.float32)

    def row(i, _):
        for c in range(H // 16):
            buf[i, pl.ds(c * 16, 16)] = zeros
        return 0

    lax.fori_loop(0, SUB, row, 0)


def _relu_add(buf_a, buf_b):
    @plsc.parallel_loop(0, SUB, unroll=8)
    def _(i):
        for c in range(H // 16):
            sl = pl.ds(c * 16, 16)
            buf_a[i, sl] = jnp.maximum(buf_a[i, sl] + buf_b[i, sl], 0.0)


def _sc_body(a_hbm, b_hbm, src_hbm, dst_hbm, out_hbm,
             src_v, dst_v, ab0, ab1, bb0, bb1, agg_sh,
             sa0, sa1, sb0, sb1, ss0, ss1):
    sid = lax.axis_index("s")
    wid = sid

    abuf = (ab0, ab1)
    bbuf = (bb0, bb1)
    sema = (sa0, sa1)
    semb = (sb0, sb1)
    sems = (ss0, ss1)

    _zero_buf(ab0)
    for kk in range(ROWS_PER_TILE // SUB):
        pltpu.sync_copy(
            ab0, agg_sh.at[pl.ds(sid * ROWS_PER_TILE + kk * SUB, SUB)])
    plsc.subcore_barrier()

    def start(j, k):
        pltpu.async_copy(a_hbm.at[src_v.at[j]], abuf[k], sema[k])
        pltpu.async_copy(b_hbm.at[dst_v.at[j]], bbuf[k], semb[k])

    def wait(j, k):
        pltpu.make_async_copy(a_hbm.at[src_v.at[j]], abuf[k], sema[k]).wait()
        pltpu.make_async_copy(b_hbm.at[dst_v.at[j]], bbuf[k], semb[k]).wait()

    def scatter_wait(k):
        pltpu.make_async_copy(abuf[k], agg_sh.at[dst_v.at[0]],
                              sems[k]).wait()

    def step(j, k, j_next, drain):
        if j_next is not None:
            if drain:
                scatter_wait(1 - k)
            start(j_next, 1 - k)
        wait(j, k)
        _relu_add(abuf[k], bbuf[k])
        pltpu.async_copy(abuf[k], agg_sh.at[dst_v.at[j]], sems[k], add=True)

    def block(kb, _):
        pltpu.sync_copy(src_hbm.at[wid, pl.ds(kb * KI, KI)], src_v)
        pltpu.sync_copy(dst_hbm.at[wid, pl.ds(kb * KI, KI)], dst_v)

        start(0, 0)
        step(0, 0, 1, False)

        def pair(j2, _):
            j = 1 + j2 * 2
            step(j, 1, j + 1, True)
            step(j + 1, 0, j + 2, True)
            return 0

        lax.fori_loop(0, (KI - 2) // 2, pair, 0)
        step(KI - 1, 1, None, False)
        scatter_wait(0)
        scatter_wait(1)
        return 0

    lax.fori_loop(0, NBLK, block, 0)

    plsc.subcore_barrier()
    base = sid * ROWS_PER_TILE
    pltpu.sync_copy(agg_sh.at[pl.ds(base, ROWS_PER_TILE)],
                    out_hbm.at[pl.ds(base, ROWS_PER_TILE)])


def _sc_aggregate(a_tab, b_tab, src_r, dst_r):
    mesh = plsc.VectorSubcoreMesh(core_axis_name="c", subcore_axis_name="s",
                                  num_cores=NC)
    fn = functools.partial(
        pl.kernel,
        out_type=jax.ShapeDtypeStruct((NP, H), jnp.float32),
        mesh=mesh,
        scratch_types=[
            pltpu.VMEM((KI, SUB), jnp.int32),
            pltpu.VMEM((KI, SUB), jnp.int32),
            pltpu.VMEM((SUB, H), jnp.float32),
            pltpu.VMEM((SUB, H), jnp.float32),
            pltpu.VMEM((SUB, H), jnp.float32),
            pltpu.VMEM((SUB, H), jnp.float32),
            pltpu.VMEM_SHARED((NP, H), jnp.float32),
            pltpu.SemaphoreType.DMA,
            pltpu.SemaphoreType.DMA,
            pltpu.SemaphoreType.DMA,
            pltpu.SemaphoreType.DMA,
            pltpu.SemaphoreType.DMA,
            pltpu.SemaphoreType.DMA,
        ],
    )(_sc_body)
    return fn(a_tab, b_tab, src_r, dst_r)


def _post_body(h_ref, agg_ref, w1a_ref, w1b_ref, b1_ref,
               w2_ref, b2_ref, w3_ref, b3_ref, o_ref):
    agg = agg_ref[...]
    h1 = jnp.tanh(
        jnp.dot(h_ref[...], w1a_ref[...], preferred_element_type=jnp.float32)
        + jnp.dot(agg, w1b_ref[...], preferred_element_type=jnp.float32)
        + b1_ref[...])
    h2 = jnp.tanh(
        jnp.dot(h1, w2_ref[...], preferred_element_type=jnp.float32)
        + b2_ref[...])
    o_ref[...] = (jnp.dot(h2, w3_ref[...], preferred_element_type=jnp.float32)
                  + b3_ref[...])


def _post_mlp(hidden, agg, w1a, w1b, b1, w2, b2, w3, b3):
    blk = 2000
    row_spec = pl.BlockSpec((blk, H), lambda i: (i, 0))
    w_spec = pl.BlockSpec((H, H), lambda i: (0, 0))
    b_spec = pl.BlockSpec((1, H), lambda i: (0, 0))
    return pl.pallas_call(
        _post_body,
        grid=(N // blk,),
        in_specs=[row_spec, row_spec,
                  w_spec, w_spec, b_spec, w_spec, b_spec, w_spec, b_spec],
        out_specs=row_spec,
        out_shape=jax.ShapeDtypeStruct((N, H), jnp.float32),
    )(hidden, agg, w1a, w1b, b1, w2, b2, w3, b3)


@jax.jit
def kernel(hidden, edge_index, We, be, W1, b1, W2, b2, W3, b3):
    ei = edge_index.astype(jnp.int32)
    pad_e = EP - E
    src_r = jnp.concatenate(
        [ei[0], jnp.full((pad_e,), N, jnp.int32)]).reshape(NW, NSUB, SUB)
    dst_r = jnp.concatenate(
        [ei[1], jnp.full((pad_e,), N, jnp.int32)]).reshape(NW, NSUB, SUB)
    hidden_pad = jnp.concatenate(
        [hidden, jnp.zeros((NP - N, H), jnp.float32)], axis=0)

    a_tab, b_tab = _pre_ab(hidden_pad, We[:H], We[H:], be.reshape(1, H))
    agg = _sc_aggregate(a_tab, b_tab, src_r, dst_r)
    return _post_mlp(hidden, agg,
                     W1[:H], W1[H:], b1.reshape(1, H),
                     W2, b2.reshape(1, H), W3, b3.reshape(1, H))


# speedup vs baseline: 1.3803x; 1.3803x over previous
import functools

import jax
import jax.numpy as jnp
from jax import lax
from jax.experimental import pallas as pl
from jax.experimental.pallas import tpu as pltpu
from jax.experimental.pallas import tpu_sc as plsc

N = 10000
E = 320000
H = 128
HC = 64

NP = 10240
NC = 2
NS = 16
SUB = 128
KI = 40
NBLK = 4
NSUB = KI * NBLK
EP = NS * NSUB * SUB
ROWS_PER_TILE = NP // NS


def _pre_body(h_ref, wa_ref, wb_ref, be_ref, a_ref, b_ref):
    h = h_ref[...]
    a_ref[...] = jnp.dot(h, wa_ref[...], preferred_element_type=jnp.float32)
    b_ref[...] = (jnp.dot(h, wb_ref[...], preferred_element_type=jnp.float32)
                  + be_ref[...])


def _pre_ab(hidden_pad, wa, wb, be2):
    blk = NP // 5
    return pl.pallas_call(
        _pre_body,
        grid=(5,),
        in_specs=[
            pl.BlockSpec((blk, H), lambda i: (i, 0)),
            pl.BlockSpec((H, H), lambda i: (0, 0)),
            pl.BlockSpec((H, H), lambda i: (0, 0)),
            pl.BlockSpec((1, H), lambda i: (0, 0)),
        ],
        out_specs=[
            pl.BlockSpec((blk, H), lambda i: (i, 0)),
            pl.BlockSpec((blk, H), lambda i: (i, 0)),
        ],
        out_shape=[
            jax.ShapeDtypeStruct((NP, H), jnp.float32),
            jax.ShapeDtypeStruct((NP, H), jnp.float32),
        ],
    )(hidden_pad, wa, wb, be2)


def _zero_buf(buf):
    zeros = jnp.zeros((16,), jnp.float32)

    def row(i, _):
        for c in range(HC // 16):
            buf[i, pl.ds(c * 16, 16)] = zeros
        return 0

    lax.fori_loop(0, SUB, row, 0)


def _relu_add(buf_a, buf_b):
    @plsc.parallel_loop(0, SUB, unroll=8)
    def _(i):
        for c in range(HC // 16):
            sl = pl.ds(c * 16, 16)
            buf_a[i, sl] = jnp.maximum(buf_a[i, sl] + buf_b[i, sl], 0.0)


def _sc_body(a_hbm, b_hbm, src_hbm, dst_hbm, out_hbm,
             src_v, dst_v, ab0, ab1, bb0, bb1, agg_sh,
             sa0, sa1, sb0, sb1, ss0, ss1):
    cid = lax.axis_index("c")
    sid = lax.axis_index("s")
    wid = sid

    abuf = (ab0, ab1)
    bbuf = (bb0, bb1)
    sema = (sa0, sa1)
    semb = (sb0, sb1)
    sems = (ss0, ss1)

    a_tab = a_hbm.at[cid]
    b_tab = b_hbm.at[cid]

    _zero_buf(ab0)
    for kk in range(ROWS_PER_TILE // SUB):
        pltpu.sync_copy(
            ab0, agg_sh.at[pl.ds(sid * ROWS_PER_TILE + kk * SUB, SUB)])
    plsc.subcore_barrier()

    def start(j, k):
        pltpu.async_copy(a_tab.at[src_v.at[j]], abuf[k], sema[k])
        pltpu.async_copy(b_tab.at[dst_v.at[j]], bbuf[k], semb[k])

    def wait(j, k):
        pltpu.make_async_copy(a_tab.at[src_v.at[j]], abuf[k], sema[k]).wait()
        pltpu.make_async_copy(b_tab.at[dst_v.at[j]], bbuf[k], semb[k]).wait()

    def scatter_wait(k):
        pltpu.make_async_copy(abuf[k], agg_sh.at[dst_v.at[0]],
                              sems[k]).wait()

    def step(j, k, j_next, drain):
        if j_next is not None:
            if drain:
                scatter_wait(1 - k)
            start(j_next, 1 - k)
        wait(j, k)
        _relu_add(abuf[k], bbuf[k])
        pltpu.async_copy(abuf[k], agg_sh.at[dst_v.at[j]], sems[k], add=True)

    def block(kb, _):
        pltpu.sync_copy(src_hbm.at[wid, pl.ds(kb * KI, KI)], src_v)
        pltpu.sync_copy(dst_hbm.at[wid, pl.ds(kb * KI, KI)], dst_v)

        start(0, 0)
        step(0, 0, 1, False)

        def pair(j2, _):
            j = 1 + j2 * 2
            step(j, 1, j + 1, True)
            step(j + 1, 0, j + 2, True)
            return 0

        lax.fori_loop(0, (KI - 2) // 2, pair, 0)
        step(KI - 1, 1, None, False)
        scatter_wait(0)
        scatter_wait(1)
        return 0

    lax.fori_loop(0, NBLK, block, 0)

    plsc.subcore_barrier()
    base = sid * ROWS_PER_TILE
    pltpu.sync_copy(agg_sh.at[pl.ds(base, ROWS_PER_TILE)],
                    out_hbm.at[cid, pl.ds(base, ROWS_PER_TILE)])


def _sc_aggregate(a_tab, b_tab, src_r, dst_r):
    mesh = plsc.VectorSubcoreMesh(core_axis_name="c", subcore_axis_name="s",
                                  num_cores=NC)
    fn = functools.partial(
        pl.kernel,
        out_type=jax.ShapeDtypeStruct((NC, NP, HC), jnp.float32),
        mesh=mesh,
        compiler_params=pltpu.CompilerParams(use_tc_tiling_on_sc=False),
        scratch_types=[
            pltpu.VMEM((KI, SUB), jnp.int32),
            pltpu.VMEM((KI, SUB), jnp.int32),
            pltpu.VMEM((SUB, HC), jnp.float32),
            pltpu.VMEM((SUB, HC), jnp.float32),
            pltpu.VMEM((SUB, HC), jnp.float32),
            pltpu.VMEM((SUB, HC), jnp.float32),
            pltpu.VMEM_SHARED((NP, HC), jnp.float32),
            pltpu.SemaphoreType.DMA,
            pltpu.SemaphoreType.DMA,
            pltpu.SemaphoreType.DMA,
            pltpu.SemaphoreType.DMA,
            pltpu.SemaphoreType.DMA,
            pltpu.SemaphoreType.DMA,
        ],
    )(_sc_body)
    return fn(a_tab, b_tab, src_r, dst_r)


def _post_body(h_ref, p0_ref, p1_ref, w1a_ref, w1b0_ref, w1b1_ref, b1_ref,
               w2_ref, b2_ref, w3_ref, b3_ref, o_ref):
    h1 = jnp.tanh(
        jnp.dot(h_ref[...], w1a_ref[...], preferred_element_type=jnp.float32)
        + jnp.dot(p0_ref[...], w1b0_ref[...],
                  preferred_element_type=jnp.float32)
        + jnp.dot(p1_ref[...], w1b1_ref[...],
                  preferred_element_type=jnp.float32)
        + b1_ref[...])
    h2 = jnp.tanh(
        jnp.dot(h1, w2_ref[...], preferred_element_type=jnp.float32)
        + b2_ref[...])
    o_ref[...] = (jnp.dot(h2, w3_ref[...], preferred_element_type=jnp.float32)
                  + b3_ref[...])


def _post_mlp(hidden, p0, p1, w1a, w1b0, w1b1, b1, w2, b2, w3, b3):
    blk = 2000
    row_spec = pl.BlockSpec((blk, H), lambda i: (i, 0))
    half_spec = pl.BlockSpec((blk, HC), lambda i: (i, 0))
    w_spec = pl.BlockSpec((H, H), lambda i: (0, 0))
    wh_spec = pl.BlockSpec((HC, H), lambda i: (0, 0))
    b_spec = pl.BlockSpec((1, H), lambda i: (0, 0))
    return pl.pallas_call(
        _post_body,
        grid=(N // blk,),
        in_specs=[row_spec, half_spec, half_spec,
                  w_spec, wh_spec, wh_spec, b_spec,
                  w_spec, b_spec, w_spec, b_spec],
        out_specs=row_spec,
        out_shape=jax.ShapeDtypeStruct((N, H), jnp.float32),
    )(hidden, p0, p1, w1a, w1b0, w1b1, b1, w2, b2, w3, b3)


@jax.jit
def kernel(hidden, edge_index, We, be, W1, b1, W2, b2, W3, b3):
    ei = edge_index.astype(jnp.int32)
    pad_e = EP - E
    src_r = jnp.concatenate(
        [ei[0], jnp.full((pad_e,), N, jnp.int32)]).reshape(NS, NSUB, SUB)
    dst_r = jnp.concatenate(
        [ei[1], jnp.full((pad_e,), N, jnp.int32)]).reshape(NS, NSUB, SUB)
    hidden_pad = jnp.concatenate(
        [hidden, jnp.zeros((NP - N, H), jnp.float32)], axis=0)

    a_full, b_full = _pre_ab(hidden_pad, We[:H], We[H:], be.reshape(1, H))
    a_tab = jnp.stack([a_full[:, :HC], a_full[:, HC:]])
    b_tab = jnp.stack([b_full[:, :HC], b_full[:, HC:]])
    parts = _sc_aggregate(a_tab, b_tab, src_r, dst_r)
    w1b = W1[H:]
    return _post_mlp(hidden, parts[0], parts[1],
                     W1[:H], w1b[:HC], w1b[HC:], b1.reshape(1, H),
                     W2, b2.reshape(1, H), W3, b3.reshape(1, H))


# speedup vs baseline: 2.2665x; 1.6420x over previous
import functools

import numpy as np
import jax
import jax.numpy as jnp
from jax import lax
from jax.experimental import pallas as pl
from jax.experimental.pallas import tpu as pltpu
from jax.experimental.pallas import tpu_sc as plsc

N = 10000
E = 320000
H = 128
HC = 64

NP = 10240
NC = 2
NS = 16
SUB = 128
KI = 40
NBLK = 4
NSUB = KI * NBLK
EP = NS * NSUB * SUB
ROWS_PER_TILE = NP // NS

_HALF_PERM = np.concatenate([
    np.arange(0, 32, 2), np.arange(1, 32, 2),
    np.arange(32, 64, 2), np.arange(33, 64, 2),
])


def _pre_body(h_ref, wa_ref, wb_ref, be_ref, a_ref, b_ref):
    h = h_ref[...]
    a_ref[...] = jnp.dot(
        h, wa_ref[...], preferred_element_type=jnp.float32
    ).astype(jnp.bfloat16)
    b_ref[...] = (jnp.dot(h, wb_ref[...], preferred_element_type=jnp.float32)
                  + be_ref[...]).astype(jnp.bfloat16)


def _pre_ab(hidden_pad, wa, wb, be2):
    blk = NP // 5
    return pl.pallas_call(
        _pre_body,
        grid=(5,),
        in_specs=[
            pl.BlockSpec((blk, H), lambda i: (i, 0)),
            pl.BlockSpec((H, H), lambda i: (0, 0)),
            pl.BlockSpec((H, H), lambda i: (0, 0)),
            pl.BlockSpec((1, H), lambda i: (0, 0)),
        ],
        out_specs=[
            pl.BlockSpec((blk, H), lambda i: (i, 0)),
            pl.BlockSpec((blk, H), lambda i: (i, 0)),
        ],
        out_shape=[
            jax.ShapeDtypeStruct((NP, H), jnp.bfloat16),
            jax.ShapeDtypeStruct((NP, H), jnp.bfloat16),
        ],
    )(hidden_pad, wa, wb, be2)


def _zero_buf(buf):
    zeros = jnp.zeros((16,), jnp.float32)

    def row(i, _):
        for c in range(HC // 16):
            buf[i, pl.ds(c * 16, 16)] = zeros
        return 0

    lax.fori_loop(0, SUB, row, 0)


def _relu_add(buf_a, buf_b, buf_o):
    hi_mask = jnp.full((16,), -65536, jnp.int32)

    @plsc.parallel_loop(0, SUB, unroll=4)
    def _(i):
        for c in range(HC // 32):
            wa = plsc.bitcast(buf_a[i, pl.ds(c * 32, 32)], jnp.int32)
            wb = plsc.bitcast(buf_b[i, pl.ds(c * 32, 32)], jnp.int32)
            ae = plsc.bitcast(wa << 16, jnp.float32)
            be = plsc.bitcast(wb << 16, jnp.float32)
            ao = plsc.bitcast(wa & hi_mask, jnp.float32)
            bo = plsc.bitcast(wb & hi_mask, jnp.float32)
            buf_o[i, pl.ds(c * 32, 16)] = jnp.maximum(ae + be, 0.0)
            buf_o[i, pl.ds(c * 32 + 16, 16)] = jnp.maximum(ao + bo, 0.0)


def _sc_body(a_hbm, b_hbm, src_hbm, dst_hbm, out_hbm,
             src_v, dst_v, ab0, ab1, bb0, bb1, fb0, fb1, agg_sh,
             sa0, sa1, sb0, sb1, ss0, ss1):
    cid = lax.axis_index("c")
    sid = lax.axis_index("s")
    wid = sid

    abuf = (ab0, ab1)
    bbuf = (bb0, bb1)
    fbuf = (fb0, fb1)
    sema = (sa0, sa1)
    semb = (sb0, sb1)
    sems = (ss0, ss1)

    a_tab = a_hbm.at[cid]
    b_tab = b_hbm.at[cid]

    _zero_buf(fb0)
    for kk in range(ROWS_PER_TILE // SUB):
        pltpu.sync_copy(
            fb0, agg_sh.at[pl.ds(sid * ROWS_PER_TILE + kk * SUB, SUB)])
    plsc.subcore_barrier()

    def start(j, k):
        pltpu.async_copy(a_tab.at[src_v.at[j]], abuf[k], sema[k])
        pltpu.async_copy(b_tab.at[dst_v.at[j]], bbuf[k], semb[k])

    def wait(j, k):
        pltpu.make_async_copy(a_tab.at[src_v.at[j]], abuf[k], sema[k]).wait()
        pltpu.make_async_copy(b_tab.at[dst_v.at[j]], bbuf[k], semb[k]).wait()

    def scatter_wait(k):
        pltpu.make_async_copy(fbuf[k], agg_sh.at[dst_v.at[0]],
                              sems[k]).wait()

    def step(j, k, j_next, drain):
        if j_next is not None:
            start(j_next, 1 - k)
        wait(j, k)
        if drain:
            scatter_wait(k)
        _relu_add(abuf[k], bbuf[k], fbuf[k])
        pltpu.async_copy(fbuf[k], agg_sh.at[dst_v.at[j]], sems[k], add=True)

    def block(kb, _):
        pltpu.sync_copy(src_hbm.at[wid, pl.ds(kb * KI, KI)], src_v)
        pltpu.sync_copy(dst_hbm.at[wid, pl.ds(kb * KI, KI)], dst_v)

        start(0, 0)
        step(0, 0, 1, False)
        step(1, 1, 2, False)

        def pair(j2, _):
            j = 2 + j2 * 2
            step(j, 0, j + 1, True)
            step(j + 1, 1, j + 2, True)
            return 0

        lax.fori_loop(0, (KI - 4) // 2, pair, 0)
        step(KI - 2, 0, KI - 1, True)
        step(KI - 1, 1, None, True)
        scatter_wait(0)
        scatter_wait(1)
        return 0

    lax.fori_loop(0, NBLK, block, 0)

    plsc.subcore_barrier()
    base = sid * ROWS_PER_TILE
    pltpu.sync_copy(agg_sh.at[pl.ds(base, ROWS_PER_TILE)],
                    out_hbm.at[cid, pl.ds(base, ROWS_PER_TILE)])


def _sc_aggregate(a_tab, b_tab, src_r, dst_r):
    mesh = plsc.VectorSubcoreMesh(core_axis_name="c", subcore_axis_name="s",
                                  num_cores=NC)
    fn = functools.partial(
        pl.kernel,
        out_type=jax.ShapeDtypeStruct((NC, NP, HC), jnp.float32),
        mesh=mesh,
        compiler_params=pltpu.CompilerParams(use_tc_tiling_on_sc=False,
                                             needs_layout_passes=False),
        scratch_types=[
            pltpu.VMEM((KI, SUB), jnp.int32),
            pltpu.VMEM((KI, SUB), jnp.int32),
            pltpu.VMEM((SUB, HC), jnp.bfloat16),
            pltpu.VMEM((SUB, HC), jnp.bfloat16),
            pltpu.VMEM((SUB, HC), jnp.bfloat16),
            pltpu.VMEM((SUB, HC), jnp.bfloat16),
            pltpu.VMEM((SUB, HC), jnp.float32),
            pltpu.VMEM((SUB, HC), jnp.float32),
            pltpu.VMEM_SHARED((NP, HC), jnp.float32),
            pltpu.SemaphoreType.DMA,
            pltpu.SemaphoreType.DMA,
            pltpu.SemaphoreType.DMA,
            pltpu.SemaphoreType.DMA,
            pltpu.SemaphoreType.DMA,
            pltpu.SemaphoreType.DMA,
        ],
    )(_sc_body)
    return fn(a_tab, b_tab, src_r, dst_r)


def _post_body(h_ref, p0_ref, p1_ref, w1a_ref, w1b0_ref, w1b1_ref, b1_ref,
               w2_ref, b2_ref, w3_ref, b3_ref, o_ref):
    h1 = jnp.tanh(
        jnp.dot(h_ref[...], w1a_ref[...], preferred_element_type=jnp.float32)
        + jnp.dot(p0_ref[...], w1b0_ref[...],
                  preferred_element_type=jnp.float32)
        + jnp.dot(p1_ref[...], w1b1_ref[...],
                  preferred_element_type=jnp.float32)
        + b1_ref[...])
    h2 = jnp.tanh(
        jnp.dot(h1, w2_ref[...], preferred_element_type=jnp.float32)
        + b2_ref[...])
    o_ref[...] = (jnp.dot(h2, w3_ref[...], preferred_element_type=jnp.float32)
                  + b3_ref[...])


def _post_mlp(hidden, p0, p1, w1a, w1b0, w1b1, b1, w2, b2, w3, b3):
    blk = 2000
    row_spec = pl.BlockSpec((blk, H), lambda i: (i, 0))
    half_spec = pl.BlockSpec((blk, HC), lambda i: (i, 0))
    w_spec = pl.BlockSpec((H, H), lambda i: (0, 0))
    wh_spec = pl.BlockSpec((HC, H), lambda i: (0, 0))
    b_spec = pl.BlockSpec((1, H), lambda i: (0, 0))
    return pl.pallas_call(
        _post_body,
        grid=(N // blk,),
        in_specs=[row_spec, half_spec, half_spec,
                  w_spec, wh_spec, wh_spec, b_spec,
                  w_spec, b_spec, w_spec, b_spec],
        out_specs=row_spec,
        out_shape=jax.ShapeDtypeStruct((N, H), jnp.float32),
    )(hidden, p0, p1, w1a, w1b0, w1b1, b1, w2, b2, w3, b3)


@jax.jit
def kernel(hidden, edge_index, We, be, W1, b1, W2, b2, W3, b3):
    ei = edge_index.astype(jnp.int32)
    pad_e = EP - E
    src_r = jnp.concatenate(
        [ei[0], jnp.full((pad_e,), N, jnp.int32)]).reshape(NS, NSUB, SUB)
    dst_r = jnp.concatenate(
        [ei[1], jnp.full((pad_e,), N, jnp.int32)]).reshape(NS, NSUB, SUB)
    hidden_pad = jnp.concatenate(
        [hidden, jnp.zeros((NP - N, H), jnp.float32)], axis=0)

    a_full, b_full = _pre_ab(hidden_pad, We[:H], We[H:], be.reshape(1, H))
    a_tab = jnp.stack([a_full[:, :HC], a_full[:, HC:]])
    b_tab = jnp.stack([b_full[:, :HC], b_full[:, HC:]])
    parts = _sc_aggregate(a_tab, b_tab, src_r, dst_r)
    w1b = W1[H:]
    return _post_mlp(hidden, parts[0], parts[1],
                     W1[:H], w1b[:HC][_HALF_PERM], w1b[HC:][_HALF_PERM],
                     b1.reshape(1, H),
                     W2, b2.reshape(1, H), W3, b3.reshape(1, H))


# speedup vs baseline: 2.2884x; 1.0097x over previous
import functools

import numpy as np
import jax
import jax.numpy as jnp
from jax import lax
from jax.experimental import pallas as pl
from jax.experimental.pallas import tpu as pltpu
from jax.experimental.pallas import tpu_sc as plsc

N = 10000
E = 320000
H = 128
HC = 64

NP = 10240
NC = 2
NS = 16
SUB = 128
KI = 40
NBLK = 4
NSUB = KI * NBLK
EP = NS * NSUB * SUB
ROWS_PER_TILE = NP // NS

_HALF_PERM = np.concatenate([
    np.arange(0, 32, 2), np.arange(1, 32, 2),
    np.arange(32, 64, 2), np.arange(33, 64, 2),
])


def _pre_body(h_ref, wa_ref, wb_ref, be_ref, a_ref, b_ref):
    h = h_ref[...]
    a = jnp.dot(h, wa_ref[...],
                preferred_element_type=jnp.float32).astype(jnp.bfloat16)
    b = (jnp.dot(h, wb_ref[...], preferred_element_type=jnp.float32)
         + be_ref[...]).astype(jnp.bfloat16)
    a_ref[0] = a[:, :HC]
    a_ref[1] = a[:, HC:]
    b_ref[0] = b[:, :HC]
    b_ref[1] = b[:, HC:]


def _pre_ab(hidden_pad, wa, wb, be2):
    blk = NP // 5
    return pl.pallas_call(
        _pre_body,
        grid=(5,),
        in_specs=[
            pl.BlockSpec((blk, H), lambda i: (i, 0)),
            pl.BlockSpec((H, H), lambda i: (0, 0)),
            pl.BlockSpec((H, H), lambda i: (0, 0)),
            pl.BlockSpec((1, H), lambda i: (0, 0)),
        ],
        out_specs=[
            pl.BlockSpec((NC, blk, HC), lambda i: (0, i, 0)),
            pl.BlockSpec((NC, blk, HC), lambda i: (0, i, 0)),
        ],
        out_shape=[
            jax.ShapeDtypeStruct((NC, NP, HC), jnp.bfloat16),
            jax.ShapeDtypeStruct((NC, NP, HC), jnp.bfloat16),
        ],
    )(hidden_pad, wa, wb, be2)


def _zero_buf(buf):
    zeros = jnp.zeros((16,), jnp.float32)

    def row(i, _):
        for c in range(HC // 16):
            buf[i, pl.ds(c * 16, 16)] = zeros
        return 0

    lax.fori_loop(0, SUB, row, 0)


def _relu_add(buf_a, buf_b, buf_o):
    hi_mask = jnp.full((16,), -65536, jnp.int32)

    @plsc.parallel_loop(0, SUB, unroll=4)
    def _(i):
        for c in range(HC // 32):
            wa = plsc.bitcast(buf_a[i, pl.ds(c * 32, 32)], jnp.int32)
            wb = plsc.bitcast(buf_b[i, pl.ds(c * 32, 32)], jnp.int32)
            ae = plsc.bitcast(wa << 16, jnp.float32)
            be = plsc.bitcast(wb << 16, jnp.float32)
            ao = plsc.bitcast(wa & hi_mask, jnp.float32)
            bo = plsc.bitcast(wb & hi_mask, jnp.float32)
            buf_o[i, pl.ds(c * 32, 16)] = jnp.maximum(ae + be, 0.0)
            buf_o[i, pl.ds(c * 32 + 16, 16)] = jnp.maximum(ao + bo, 0.0)


def _sc_body(a_hbm, b_hbm, src_hbm, dst_hbm, out_hbm,
             src_v, dst_v, ab0, ab1, bb0, bb1, fb0, fb1, agg_sh,
             sa0, sa1, sb0, sb1, ss0, ss1):
    cid = lax.axis_index("c")
    sid = lax.axis_index("s")
    wid = sid

    abuf = (ab0, ab1)
    bbuf = (bb0, bb1)
    fbuf = (fb0, fb1)
    sema = (sa0, sa1)
    semb = (sb0, sb1)
    sems = (ss0, ss1)

    a_tab = a_hbm.at[cid]
    b_tab = b_hbm.at[cid]

    _zero_buf(fb0)
    for kk in range(ROWS_PER_TILE // SUB):
        pltpu.sync_copy(
            fb0, agg_sh.at[pl.ds(sid * ROWS_PER_TILE + kk * SUB, SUB)])
    plsc.subcore_barrier()

    def start(j, k):
        pltpu.async_copy(a_tab.at[src_v.at[j]], abuf[k], sema[k])
        pltpu.async_copy(b_tab.at[dst_v.at[j]], bbuf[k], semb[k])

    def wait(j, k):
        pltpu.make_async_copy(a_tab.at[src_v.at[j]], abuf[k], sema[k]).wait()
        pltpu.make_async_copy(b_tab.at[dst_v.at[j]], bbuf[k], semb[k]).wait()

    def scatter_wait(k):
        pltpu.make_async_copy(fbuf[k], agg_sh.at[dst_v.at[0]],
                              sems[k]).wait()

    def step(j, k, j_next, drain):
        if j_next is not None:
            start(j_next, 1 - k)
        wait(j, k)
        if drain:
            scatter_wait(k)
        _relu_add(abuf[k], bbuf[k], fbuf[k])
        pltpu.async_copy(fbuf[k], agg_sh.at[dst_v.at[j]], sems[k], add=True)

    def block(kb, _):
        pltpu.sync_copy(src_hbm.at[wid, pl.ds(kb * KI, KI)], src_v)
        pltpu.sync_copy(dst_hbm.at[wid, pl.ds(kb * KI, KI)], dst_v)

        start(0, 0)
        step(0, 0, 1, False)
        step(1, 1, 2, False)

        def pair(j2, _):
            j = 2 + j2 * 2
            step(j, 0, j + 1, True)
            step(j + 1, 1, j + 2, True)
            return 0

        lax.fori_loop(0, (KI - 4) // 2, pair, 0)
        step(KI - 2, 0, KI - 1, True)
        step(KI - 1, 1, None, True)
        scatter_wait(0)
        scatter_wait(1)
        return 0

    lax.fori_loop(0, NBLK, block, 0)

    plsc.subcore_barrier()
    base = sid * ROWS_PER_TILE
    pltpu.sync_copy(agg_sh.at[pl.ds(base, ROWS_PER_TILE)],
                    out_hbm.at[cid, pl.ds(base, ROWS_PER_TILE)])


def _sc_aggregate(a_tab, b_tab, src_r, dst_r):
    mesh = plsc.VectorSubcoreMesh(core_axis_name="c", subcore_axis_name="s",
                                  num_cores=NC)
    fn = functools.partial(
        pl.kernel,
        out_type=jax.ShapeDtypeStruct((NC, NP, HC), jnp.float32),
        mesh=mesh,
        compiler_params=pltpu.CompilerParams(use_tc_tiling_on_sc=False,
                                             needs_layout_passes=False),
        scratch_types=[
            pltpu.VMEM((KI, SUB), jnp.int32),
            pltpu.VMEM((KI, SUB), jnp.int32),
            pltpu.VMEM((SUB, HC), jnp.bfloat16),
            pltpu.VMEM((SUB, HC), jnp.bfloat16),
            pltpu.VMEM((SUB, HC), jnp.bfloat16),
            pltpu.VMEM((SUB, HC), jnp.bfloat16),
            pltpu.VMEM((SUB, HC), jnp.float32),
            pltpu.VMEM((SUB, HC), jnp.float32),
            pltpu.VMEM_SHARED((NP, HC), jnp.float32),
            pltpu.SemaphoreType.DMA,
            pltpu.SemaphoreType.DMA,
            pltpu.SemaphoreType.DMA,
            pltpu.SemaphoreType.DMA,
            pltpu.SemaphoreType.DMA,
            pltpu.SemaphoreType.DMA,
        ],
    )(_sc_body)
    return fn(a_tab, b_tab, src_r, dst_r)


def _post_body(h_ref, p0_ref, p1_ref, w1a_ref, w1b0_ref, w1b1_ref, b1_ref,
               w2_ref, b2_ref, w3_ref, b3_ref, o_ref):
    h1 = jnp.tanh(
        jnp.dot(h_ref[...], w1a_ref[...], preferred_element_type=jnp.float32)
        + jnp.dot(p0_ref[...], w1b0_ref[...],
                  preferred_element_type=jnp.float32)
        + jnp.dot(p1_ref[...], w1b1_ref[...],
                  preferred_element_type=jnp.float32)
        + b1_ref[...])
    h2 = jnp.tanh(
        jnp.dot(h1, w2_ref[...], preferred_element_type=jnp.float32)
        + b2_ref[...])
    o_ref[...] = (jnp.dot(h2, w3_ref[...], preferred_element_type=jnp.float32)
                  + b3_ref[...])


def _post_mlp(hidden, p0, p1, w1a, w1b0, w1b1, b1, w2, b2, w3, b3):
    blk = 2000
    row_spec = pl.BlockSpec((blk, H), lambda i: (i, 0))
    half_spec = pl.BlockSpec((blk, HC), lambda i: (i, 0))
    w_spec = pl.BlockSpec((H, H), lambda i: (0, 0))
    wh_spec = pl.BlockSpec((HC, H), lambda i: (0, 0))
    b_spec = pl.BlockSpec((1, H), lambda i: (0, 0))
    return pl.pallas_call(
        _post_body,
        grid=(N // blk,),
        in_specs=[row_spec, half_spec, half_spec,
                  w_spec, wh_spec, wh_spec, b_spec,
                  w_spec, b_spec, w_spec, b_spec],
        out_specs=row_spec,
        out_shape=jax.ShapeDtypeStruct((N, H), jnp.float32),
    )(hidden, p0, p1, w1a, w1b0, w1b1, b1, w2, b2, w3, b3)


@jax.jit
def kernel(hidden, edge_index, We, be, W1, b1, W2, b2, W3, b3):
    ei = edge_index.astype(jnp.int32)
    pad_e = EP - E
    src_r = jnp.concatenate(
        [ei[0], jnp.full((pad_e,), N, jnp.int32)]).reshape(NS, NSUB, SUB)
    dst_r = jnp.concatenate(
        [ei[1], jnp.full((pad_e,), N, jnp.int32)]).reshape(NS, NSUB, SUB)
    hidden_pad = jnp.concatenate(
        [hidden, jnp.zeros((NP - N, H), jnp.float32)], axis=0)

    a_tab, b_tab = _pre_ab(hidden_pad, We[:H], We[H:], be.reshape(1, H))
    parts = _sc_aggregate(a_tab, b_tab, src_r, dst_r)
    w1b = W1[H:]
    return _post_mlp(hidden, parts[0], parts[1],
                     W1[:H], w1b[:HC][_HALF_PERM], w1b[HC:][_HALF_PERM],
                     b1.reshape(1, H),
                     W2, b2.reshape(1, H), W3, b3.reshape(1, H))
